# baseline (speedup 1.0000x reference)
import functools

import jax
import jax.numpy as jnp
from jax import lax
from jax.experimental import pallas as pl
from jax.experimental.pallas import tpu as pltpu
from jax.experimental.pallas import tpu_sc as plsc

V = 100000
D = 128
K = 40
B = 4096
NEG = 5
GAMMA = 0.001

NC = 2
NS = 16
NW = NC * NS
NSPLIT = 2
BH = B // NSPLIT
BPW = BH // NW
NCHAIN = 2
SB = BPW // NCHAIN


@functools.cache
def _make_sc_gather():
    mesh = plsc.VectorSubcoreMesh(core_axis_name="c", subcore_axis_name="s")

    @functools.partial(
        pl.kernel,
        out_type=(
            jax.ShapeDtypeStruct((BH, D), jnp.float32),
            jax.ShapeDtypeStruct((BH, D), jnp.float32),
            jax.ShapeDtypeStruct((BH, D), jnp.float32),
        ),
        mesh=mesh,
        scratch_types=[
            pltpu.VMEM((NEG, BPW), jnp.int32),
            pltpu.VMEM((2, BPW), jnp.int32),
            pltpu.VMEM((BPW, D), jnp.float32),
            pltpu.VMEM((BPW, D), jnp.float32),
            pltpu.VMEM((BPW, D), jnp.float32),
            pltpu.SemaphoreType.DMA,
            pltpu.SemaphoreType.DMA,
            pltpu.SemaphoreType.DMA,
        ] + [pltpu.SemaphoreType.DMA] * NCHAIN,
    )
    def _sc_gather(nidx_hbm, uvidx_hbm, table_hbm,
                   u_out, v_out, n_out,
                   idx_n, idx_uv, u_rows, v_rows, n_acc,
                   sem_u, sem_v, sem_i, *sem_c):
        wid = lax.axis_index("s") * NC + lax.axis_index("c")
        base = wid * BPW
        pltpu.sync_copy(nidx_hbm.at[wid], idx_n)
        ci = pltpu.async_copy(uvidx_hbm.at[wid], idx_uv, sem_i)
        chains = [
            pltpu.async_copy(table_hbm.at[idx_n.at[0, pl.ds(s * SB, SB)]],
                             n_acc.at[pl.ds(s * SB, SB)], sem_c[s])
            for s in range(NCHAIN)
        ]
        ci.wait()
        cu = pltpu.async_copy(table_hbm.at[idx_uv.at[0]], u_rows, sem_u)
        cv = pltpu.async_copy(table_hbm.at[idx_uv.at[1]], v_rows, sem_v)
        for j in range(1, NEG):
            for s in range(NCHAIN):
                chains[s].wait()
                chains[s] = pltpu.async_copy(
                    table_hbm.at[idx_n.at[j, pl.ds(s * SB, SB)]],
                    n_acc.at[pl.ds(s * SB, SB)], sem_c[s], add=True)
        cu.wait()
        wu = pltpu.async_copy(u_rows, u_out.at[pl.ds(base, BPW)], sem_u)
        cv.wait()
        wv = pltpu.async_copy(v_rows, v_out.at[pl.ds(base, BPW)], sem_v)
        wns = []
        for s in range(NCHAIN):
            chains[s].wait()
            wns.append(pltpu.async_copy(
                n_acc.at[pl.ds(s * SB, SB)],
                n_out.at[pl.ds(base + s * SB, SB)], sem_c[s]))
        wu.wait()
        wv.wait()
        for w in wns:
            w.wait()

    return _sc_gather


def _log_sigmoid(x):
    return jnp.minimum(x, 0.0) - jnp.log(1.0 + jnp.exp(-jnp.abs(x)))


CB = 1024
NCHUNK = BH // CB


def _tc_body(u_ref, v_ref, n_ref, c_ref, out_ref):
    i = pl.program_id(0)
    u = u_ref[...]
    v = v_ref[...]
    ns = n_ref[...]
    ones = jnp.ones((D, 1), jnp.float32)
    pos = lax.dot_general(u * v, ones, (((1,), (0,)), ((), ())),
                          preferred_element_type=jnp.float32)
    neg = -lax.dot_general(u * ns, ones, (((1,), (0,)), ((), ())),
                           preferred_element_type=jnp.float32)
    loss_sum = jnp.sum(_log_sigmoid(pos) + _log_sigmoid(neg))
    c = c_ref[...]
    cross_t = lax.dot_general(c, u, (((1,), (1,)), ((), ())),
                              preferred_element_type=jnp.float32)
    cnorm_t = jnp.sum(c * c, axis=1, keepdims=True)
    m = cnorm_t - 2.0 * cross_t
    loss2_sum = jnp.sum(jnp.min(m, axis=0, keepdims=True)) + jnp.sum(u * u)
    partial = -(loss_sum / B) + (GAMMA / B) * loss2_sum

    @pl.when(i == 0)
    def _():
        out_ref[...] = jnp.zeros((1, 1), jnp.float32)

    out_ref[...] += jnp.reshape(partial, (1, 1))


_tc_compute = pl.pallas_call(
    _tc_body,
    grid=(NCHUNK,),
    in_specs=[
        pl.BlockSpec((CB, D), lambda i: (i, 0)),
        pl.BlockSpec((CB, D), lambda i: (i, 0)),
        pl.BlockSpec((CB, D), lambda i: (i, 0)),
        pl.BlockSpec((K, D), lambda i: (0, 0)),
    ],
    out_specs=pl.BlockSpec((1, 1), lambda i: (0, 0)),
    out_shape=jax.ShapeDtypeStruct((1, 1), jnp.float32),
)


def kernel(u_node, v_node, negative_nodes, emb_u_weight, emb_com_weight):
    uv_idx = jnp.stack(
        [u_node.reshape(NSPLIT, NW, BPW), v_node.reshape(NSPLIT, NW, BPW)],
        axis=2).astype(jnp.int32)
    neg_idx = jnp.transpose(
        negative_nodes.reshape(NSPLIT, NW, BPW, NEG),
        (0, 1, 3, 2)).astype(jnp.int32)
    sc = _make_sc_gather()
    gathered = [sc(neg_idx[h], uv_idx[h], emb_u_weight)
                for h in range(NSPLIT)]
    total = None
    for h in range(NSPLIT):
        u_e, v_e, nsum = gathered[h]
        part = _tc_compute(u_e, v_e, nsum, emb_com_weight)
        total = part if total is None else total + part
    return total[0, 0]


# speedup vs baseline: 1.2765x; 1.2765x over previous
import functools

import jax
import jax.numpy as jnp
from jax import lax
from jax.experimental import pallas as pl
from jax.experimental.pallas import tpu as pltpu
from jax.experimental.pallas import tpu_sc as plsc

V = 100000
D = 128
K = 40
B = 4096
NEG = 5
GAMMA = 0.001

NC = 2
NS = 16
NW = NC * NS
BPW = B // NW
NCHAIN = 1
SB = BPW // NCHAIN

@functools.cache
def _make_sc_gather():
    mesh = plsc.VectorSubcoreMesh(core_axis_name="c", subcore_axis_name="s")

    @functools.partial(
        pl.kernel,
        out_type=(
            jax.ShapeDtypeStruct((B, D), jnp.float32),
            jax.ShapeDtypeStruct((B, D), jnp.float32),
            jax.ShapeDtypeStruct((B, D), jnp.float32),
        ),
        mesh=mesh,
        scratch_types=[
            pltpu.VMEM((NEG, BPW), jnp.int32),
            pltpu.VMEM((2, BPW), jnp.int32),
            pltpu.VMEM((BPW, D), jnp.float32),
            pltpu.VMEM((BPW, D), jnp.float32),
            pltpu.VMEM((BPW, D), jnp.float32),
            pltpu.SemaphoreType.DMA,
            pltpu.SemaphoreType.DMA,
            pltpu.SemaphoreType.DMA,
        ] + [pltpu.SemaphoreType.DMA] * NCHAIN,
    )
    def _sc_gather(nidx_hbm, uvidx_hbm, table_hbm,
                   u_out, v_out, n_out,
                   idx_n, idx_uv, u_rows, v_rows, n_acc,
                   sem_u, sem_v, sem_i, *sem_c):
        wid = lax.axis_index("s") * NC + lax.axis_index("c")
        base = wid * BPW
        pltpu.sync_copy(nidx_hbm.at[wid], idx_n)
        ci = pltpu.async_copy(uvidx_hbm.at[wid], idx_uv, sem_i)
        chains = [
            pltpu.async_copy(table_hbm.at[idx_n.at[0, pl.ds(s * SB, SB)]],
                             n_acc.at[pl.ds(s * SB, SB)], sem_c[s])
            for s in range(NCHAIN)
        ]
        ci.wait()
        cu = pltpu.async_copy(table_hbm.at[idx_uv.at[0]], u_rows, sem_u)
        cv = pltpu.async_copy(table_hbm.at[idx_uv.at[1]], v_rows, sem_v)
        for j in range(1, NEG):
            for s in range(NCHAIN):
                chains[s].wait()
                chains[s] = pltpu.async_copy(
                    table_hbm.at[idx_n.at[j, pl.ds(s * SB, SB)]],
                    n_acc.at[pl.ds(s * SB, SB)], sem_c[s], add=True)
        cu.wait()
        wu = pltpu.async_copy(u_rows, u_out.at[pl.ds(base, BPW)], sem_u)
        cv.wait()
        wv = pltpu.async_copy(v_rows, v_out.at[pl.ds(base, BPW)], sem_v)
        wns = []
        for s in range(NCHAIN):
            chains[s].wait()
            wns.append(pltpu.async_copy(
                n_acc.at[pl.ds(s * SB, SB)],
                n_out.at[pl.ds(base + s * SB, SB)], sem_c[s]))
        wu.wait()
        wv.wait()
        for w in wns:
            w.wait()

    return _sc_gather


def _log_sigmoid(x):
    return jnp.minimum(x, 0.0) - jnp.log(1.0 + jnp.exp(-jnp.abs(x)))


CB = 1024
NCHUNK = B // CB


def _tc_body(u_ref, v_ref, n_ref, c_ref, out_ref):
    i = pl.program_id(0)
    u = u_ref[...]
    v = v_ref[...]
    ns = n_ref[...]
    ones = jnp.ones((D, 1), jnp.float32)
    pos = lax.dot_general(u * v, ones, (((1,), (0,)), ((), ())),
                          preferred_element_type=jnp.float32)
    neg = -lax.dot_general(u * ns, ones, (((1,), (0,)), ((), ())),
                           preferred_element_type=jnp.float32)
    loss_sum = jnp.sum(_log_sigmoid(pos) + _log_sigmoid(neg))
    c = c_ref[...]
    cross_t = lax.dot_general(c, u, (((1,), (1,)), ((), ())),
                              preferred_element_type=jnp.float32)
    cnorm_t = jnp.sum(c * c, axis=1, keepdims=True)
    m = cnorm_t - 2.0 * cross_t
    loss2_sum = jnp.sum(jnp.min(m, axis=0, keepdims=True)) + jnp.sum(u * u)
    partial = -(loss_sum / B) + (GAMMA / B) * loss2_sum

    @pl.when(i == 0)
    def _():
        out_ref[...] = jnp.zeros((1, 1), jnp.float32)

    out_ref[...] += jnp.reshape(partial, (1, 1))


_tc_compute = pl.pallas_call(
    _tc_body,
    grid=(NCHUNK,),
    in_specs=[
        pl.BlockSpec((CB, D), lambda i: (i, 0)),
        pl.BlockSpec((CB, D), lambda i: (i, 0)),
        pl.BlockSpec((CB, D), lambda i: (i, 0)),
        pl.BlockSpec((K, D), lambda i: (0, 0)),
    ],
    out_specs=pl.BlockSpec((1, 1), lambda i: (0, 0)),
    out_shape=jax.ShapeDtypeStruct((1, 1), jnp.float32),
)


def kernel(u_node, v_node, negative_nodes, emb_u_weight, emb_com_weight):
    uv_idx = jnp.stack(
        [u_node.reshape(NW, BPW), v_node.reshape(NW, BPW)],
        axis=1).astype(jnp.int32)
    neg_idx = jnp.transpose(
        negative_nodes.reshape(NW, BPW, NEG), (0, 2, 1)).astype(jnp.int32)
    u_e, v_e, nsum = _make_sc_gather()(neg_idx, uv_idx, emb_u_weight)
    out = _tc_compute(u_e, v_e, nsum, emb_com_weight)
    return out[0, 0]


# speedup vs baseline: 1.3155x; 1.0306x over previous
import functools

import jax
import jax.numpy as jnp
from jax import lax
from jax.experimental import pallas as pl
from jax.experimental.pallas import tpu as pltpu
from jax.experimental.pallas import tpu_sc as plsc

V = 100000
D = 128
K = 40
B = 4096
NEG = 5
GAMMA = 0.001

NC = 2
NS = 16
NW = NC * NS
BPW = B // NW
NCHAIN = 2
SB = BPW // NCHAIN

@functools.cache
def _make_sc_gather():
    mesh = plsc.VectorSubcoreMesh(core_axis_name="c", subcore_axis_name="s")

    @functools.partial(
        pl.kernel,
        out_type=(
            jax.ShapeDtypeStruct((B, D), jnp.float32),
            jax.ShapeDtypeStruct((B, D), jnp.float32),
            jax.ShapeDtypeStruct((B, D), jnp.float32),
        ),
        mesh=mesh,
        scratch_types=[
            pltpu.VMEM((NEG, BPW), jnp.int32),
            pltpu.VMEM((2, BPW), jnp.int32),
            pltpu.VMEM((BPW, D), jnp.float32),
            pltpu.VMEM((BPW, D), jnp.float32),
            pltpu.VMEM((BPW, D), jnp.float32),
            pltpu.SemaphoreType.DMA,
            pltpu.SemaphoreType.DMA,
            pltpu.SemaphoreType.DMA,
        ] + [pltpu.SemaphoreType.DMA] * NCHAIN,
    )
    def _sc_gather(nidx_hbm, uvidx_hbm, table_hbm,
                   u_out, v_out, n_out,
                   idx_n, idx_uv, u_rows, v_rows, n_acc,
                   sem_u, sem_v, sem_i, *sem_c):
        wid = lax.axis_index("s") * NC + lax.axis_index("c")
        base = wid * BPW
        pltpu.sync_copy(nidx_hbm.at[wid], idx_n)
        ci = pltpu.async_copy(uvidx_hbm.at[wid], idx_uv, sem_i)
        chains = [
            pltpu.async_copy(table_hbm.at[idx_n.at[0, pl.ds(s * SB, SB)]],
                             n_acc.at[pl.ds(s * SB, SB)], sem_c[s])
            for s in range(NCHAIN)
        ]
        ci.wait()
        cu = pltpu.async_copy(table_hbm.at[idx_uv.at[0]], u_rows, sem_u)
        cv = pltpu.async_copy(table_hbm.at[idx_uv.at[1]], v_rows, sem_v)
        for j in range(1, NEG):
            for s in range(NCHAIN):
                chains[s].wait()
                chains[s] = pltpu.async_copy(
                    table_hbm.at[idx_n.at[j, pl.ds(s * SB, SB)]],
                    n_acc.at[pl.ds(s * SB, SB)], sem_c[s], add=True)
        cu.wait()
        wu = pltpu.async_copy(u_rows, u_out.at[pl.ds(base, BPW)], sem_u)
        cv.wait()
        wv = pltpu.async_copy(v_rows, v_out.at[pl.ds(base, BPW)], sem_v)
        wns = []
        for s in range(NCHAIN):
            chains[s].wait()
            wns.append(pltpu.async_copy(
                n_acc.at[pl.ds(s * SB, SB)],
                n_out.at[pl.ds(base + s * SB, SB)], sem_c[s]))
        wu.wait()
        wv.wait()
        for w in wns:
            w.wait()

    return _sc_gather


def _log_sigmoid(x):
    return jnp.minimum(x, 0.0) - jnp.log(1.0 + jnp.exp(-jnp.abs(x)))


CB = 1024
NCHUNK = B // CB


def _tc_body(u_ref, v_ref, n_ref, c_ref, out_ref):
    i = pl.program_id(0)
    u = u_ref[...]
    v = v_ref[...]
    ns = n_ref[...]
    onesr = jnp.ones((1, D), jnp.float32)
    pos = lax.dot_general(onesr, u * v, (((1,), (1,)), ((), ())),
                          preferred_element_type=jnp.float32)
    neg = -lax.dot_general(onesr, u * ns, (((1,), (1,)), ((), ())),
                           preferred_element_type=jnp.float32)
    loss_sum = jnp.sum(_log_sigmoid(pos) + _log_sigmoid(neg))
    c = c_ref[...]
    cross_t = lax.dot_general(c, u, (((1,), (1,)), ((), ())),
                              preferred_element_type=jnp.float32)
    cnorm_t = jnp.sum(c * c, axis=1, keepdims=True)
    m = cnorm_t - 2.0 * cross_t
    loss2_sum = jnp.sum(jnp.min(m, axis=0, keepdims=True)) + jnp.sum(u * u)
    partial = -(loss_sum / B) + (GAMMA / B) * loss2_sum

    @pl.when(i == 0)
    def _():
        out_ref[...] = jnp.zeros((1, 1), jnp.float32)

    out_ref[...] += jnp.reshape(partial, (1, 1))


_tc_compute = pl.pallas_call(
    _tc_body,
    grid=(NCHUNK,),
    in_specs=[
        pl.BlockSpec((CB, D), lambda i: (i, 0)),
        pl.BlockSpec((CB, D), lambda i: (i, 0)),
        pl.BlockSpec((CB, D), lambda i: (i, 0)),
        pl.BlockSpec((K, D), lambda i: (0, 0)),
    ],
    out_specs=pl.BlockSpec((1, 1), lambda i: (0, 0)),
    out_shape=jax.ShapeDtypeStruct((1, 1), jnp.float32),
)


def kernel(u_node, v_node, negative_nodes, emb_u_weight, emb_com_weight):
    uv_idx = jnp.stack(
        [u_node.reshape(NW, BPW), v_node.reshape(NW, BPW)],
        axis=1).astype(jnp.int32)
    neg_idx = jnp.transpose(
        negative_nodes.reshape(NW, BPW, NEG), (0, 2, 1)).astype(jnp.int32)
    u_e, v_e, nsum = _make_sc_gather()(neg_idx, uv_idx, emb_u_weight)
    out = _tc_compute(u_e, v_e, nsum, emb_com_weight)
    return out[0, 0]


# speedup vs baseline: 1.3601x; 1.0339x over previous
import functools

import jax
import jax.numpy as jnp
from jax import lax
from jax.experimental import pallas as pl
from jax.experimental.pallas import tpu as pltpu
from jax.experimental.pallas import tpu_sc as plsc

V = 100000
D = 128
K = 40
B = 4096
NEG = 5
GAMMA = 0.001

NC = 2
NS = 16
NW = NC * NS
BPW = B // NW
NCHAIN = 2
SB = BPW // NCHAIN

@functools.cache
def _make_sc_gather():
    mesh = plsc.VectorSubcoreMesh(core_axis_name="c", subcore_axis_name="s")

    @functools.partial(
        pl.kernel,
        out_type=(
            jax.ShapeDtypeStruct((B, D), jnp.float32),
            jax.ShapeDtypeStruct((B, D), jnp.float32),
            jax.ShapeDtypeStruct((B, D), jnp.float32),
        ),
        mesh=mesh,
        scratch_types=[
            pltpu.VMEM((NEG, BPW), jnp.int32),
            pltpu.VMEM((2, BPW), jnp.int32),
            pltpu.VMEM((BPW, D), jnp.float32),
            pltpu.VMEM((BPW, D), jnp.float32),
            pltpu.VMEM((BPW, D), jnp.float32),
            pltpu.SemaphoreType.DMA,
            pltpu.SemaphoreType.DMA,
            pltpu.SemaphoreType.DMA,
        ] + [pltpu.SemaphoreType.DMA] * NCHAIN,
    )
    def _sc_gather(nidx_hbm, uvidx_hbm, table_hbm,
                   u_out, v_out, n_out,
                   idx_n, idx_uv, u_rows, v_rows, n_acc,
                   sem_u, sem_v, sem_i, *sem_c):
        wid = lax.axis_index("s") * NC + lax.axis_index("c")
        base = wid * BPW
        pltpu.sync_copy(nidx_hbm.at[wid], idx_n)
        ci = pltpu.async_copy(uvidx_hbm.at[wid], idx_uv, sem_i)
        chains = [
            pltpu.async_copy(table_hbm.at[idx_n.at[0, pl.ds(s * SB, SB)]],
                             n_acc.at[pl.ds(s * SB, SB)], sem_c[s])
            for s in range(NCHAIN)
        ]
        ci.wait()
        cu = pltpu.async_copy(table_hbm.at[idx_uv.at[0]], u_rows, sem_u)
        cv = pltpu.async_copy(table_hbm.at[idx_uv.at[1]], v_rows, sem_v)
        for j in range(1, NEG):
            for s in range(NCHAIN):
                chains[s].wait()
                chains[s] = pltpu.async_copy(
                    table_hbm.at[idx_n.at[j, pl.ds(s * SB, SB)]],
                    n_acc.at[pl.ds(s * SB, SB)], sem_c[s], add=True)
        cu.wait()
        wu = pltpu.async_copy(u_rows, u_out.at[pl.ds(base, BPW)], sem_u)
        cv.wait()
        wv = pltpu.async_copy(v_rows, v_out.at[pl.ds(base, BPW)], sem_v)
        wns = []
        for s in range(NCHAIN):
            chains[s].wait()
            wns.append(pltpu.async_copy(
                n_acc.at[pl.ds(s * SB, SB)],
                n_out.at[pl.ds(base + s * SB, SB)], sem_c[s]))
        wu.wait()
        wv.wait()
        for w in wns:
            w.wait()

    return _sc_gather


def _log_sigmoid(x):
    return jnp.minimum(x, 0.0) - jnp.log(1.0 + jnp.exp(-jnp.abs(x)))


CB = 2048
NCHUNK = B // CB


def _tc_body(u_ref, v_ref, n_ref, c_ref, out_ref):
    i = pl.program_id(0)
    u = u_ref[...]
    v = v_ref[...]
    ns = n_ref[...]
    onesr = jnp.ones((1, D), jnp.float32)
    pos = lax.dot_general(onesr, u * v, (((1,), (1,)), ((), ())),
                          preferred_element_type=jnp.float32)
    neg = -lax.dot_general(onesr, u * ns, (((1,), (1,)), ((), ())),
                           preferred_element_type=jnp.float32)
    loss_sum = jnp.sum(_log_sigmoid(pos) + _log_sigmoid(neg))
    c = c_ref[...]
    cross_t = lax.dot_general(c, u, (((1,), (1,)), ((), ())),
                              preferred_element_type=jnp.float32)
    cnorm_t = jnp.sum(c * c, axis=1, keepdims=True)
    m = cnorm_t - 2.0 * cross_t
    loss2_sum = jnp.sum(jnp.min(m, axis=0, keepdims=True)) + jnp.sum(u * u)
    partial = -(loss_sum / B) + (GAMMA / B) * loss2_sum

    @pl.when(i == 0)
    def _():
        out_ref[...] = jnp.zeros((1, 1), jnp.float32)

    out_ref[...] += jnp.reshape(partial, (1, 1))


_tc_compute = pl.pallas_call(
    _tc_body,
    grid=(NCHUNK,),
    in_specs=[
        pl.BlockSpec((CB, D), lambda i: (i, 0)),
        pl.BlockSpec((CB, D), lambda i: (i, 0)),
        pl.BlockSpec((CB, D), lambda i: (i, 0)),
        pl.BlockSpec((K, D), lambda i: (0, 0)),
    ],
    out_specs=pl.BlockSpec((1, 1), lambda i: (0, 0)),
    out_shape=jax.ShapeDtypeStruct((1, 1), jnp.float32),
)


def kernel(u_node, v_node, negative_nodes, emb_u_weight, emb_com_weight):
    uv_idx = jnp.stack(
        [u_node.reshape(NW, BPW), v_node.reshape(NW, BPW)],
        axis=1).astype(jnp.int32)
    neg_idx = jnp.transpose(
        negative_nodes.reshape(NW, BPW, NEG), (0, 2, 1)).astype(jnp.int32)
    u_e, v_e, nsum = _make_sc_gather()(neg_idx, uv_idx, emb_u_weight)
    out = _tc_compute(u_e, v_e, nsum, emb_com_weight)
    return out[0, 0]
